# Initial kernel scaffold; baseline (speedup 1.0000x reference)
import jax
import jax.numpy as jnp
from jax.experimental import pallas as pl


def kernel(h, cb0, cb1, cb2, cb3):
    raise NotImplementedError("write your pallas kernel here")


# speedup vs baseline: 1.0153x; 1.0153x over previous
import functools

import jax
import jax.numpy as jnp
from jax.experimental import pallas as pl


N_LEVELS = 4


def _rvq_body(h_ref, w0_ref, w1_ref, w2_ref, w3_ref,
              zq_ref, sids_ref, res_ref, quant_ref):
    r = h_ref[...]
    bB, D = r.shape
    K = w0_ref.shape[0]
    iota_k = jax.lax.broadcasted_iota(jnp.int32, (1, K), 1)

    zq = jnp.zeros_like(r)
    idx_cols = []
    for lvl, w_ref in enumerate((w0_ref, w1_ref, w2_ref, w3_ref)):
        W = w_ref[...]
        w_sq = jnp.sum(W * W, axis=1)[None, :]
        r_sq = jnp.sum(r * r, axis=1, keepdims=True)
        s = jax.lax.dot_general(r, W, (((1,), (1,)), ((), ())),
                                preferred_element_type=jnp.float32)
        dist = r_sq - 2.0 * s + w_sq
        m = jnp.min(dist, axis=1, keepdims=True)
        idx = jnp.min(jnp.where(dist == m, iota_k, K), axis=1)

        onehot = (iota_k == idx[:, None]).astype(jnp.float32)
        w_hi = W.astype(jnp.bfloat16).astype(jnp.float32)
        w_lo = W - w_hi
        q_hi = jax.lax.dot_general(onehot, w_hi, (((1,), (0,)), ((), ())),
                                   preferred_element_type=jnp.float32)
        q_lo = jax.lax.dot_general(onehot, w_lo, (((1,), (0,)), ((), ())),
                                   preferred_element_type=jnp.float32)
        q = q_hi + q_lo

        res_ref[:, lvl, :] = r
        quant_ref[:, lvl, :] = q
        idx_cols.append(idx)
        zq = zq + q
        r = r - q

    sids_ref[...] = jnp.stack(idx_cols, axis=1)
    zq_ref[...] = zq


@jax.jit
def _rvq(h, cb0, cb1, cb2, cb3):
    B, D = h.shape
    K = cb0.shape[0]
    bB = 256
    grid = (B // bB,)
    cb_spec = pl.BlockSpec((K, D), lambda i: (0, 0))
    out_shapes = (
        jax.ShapeDtypeStruct((B, D), jnp.float32),
        jax.ShapeDtypeStruct((B, N_LEVELS), jnp.int32),
        jax.ShapeDtypeStruct((B, N_LEVELS, D), jnp.float32),
        jax.ShapeDtypeStruct((B, N_LEVELS, D), jnp.float32),
    )
    out_specs = (
        pl.BlockSpec((bB, D), lambda i: (i, 0)),
        pl.BlockSpec((bB, N_LEVELS), lambda i: (i, 0)),
        pl.BlockSpec((bB, N_LEVELS, D), lambda i: (i, 0, 0)),
        pl.BlockSpec((bB, N_LEVELS, D), lambda i: (i, 0, 0)),
    )
    in_specs = [
        pl.BlockSpec((bB, D), lambda i: (i, 0)),
        cb_spec, cb_spec, cb_spec, cb_spec,
    ]
    return pl.pallas_call(
        _rvq_body,
        grid=grid,
        in_specs=in_specs,
        out_specs=out_specs,
        out_shape=out_shapes,
    )(h, cb0, cb1, cb2, cb3)


def kernel(h, cb0, cb1, cb2, cb3):
    z_q, sids, residuals, quantized = _rvq(h, cb0, cb1, cb2, cb3)
    active_mask = jnp.ones((N_LEVELS,), dtype=h.dtype)
    return (z_q, sids, residuals, quantized, active_mask)


# speedup vs baseline: 1.2369x; 1.2183x over previous
import jax
import jax.numpy as jnp
from jax.experimental import pallas as pl
from jax.experimental.pallas import tpu as pltpu


N_LEVELS = 4


def _rvq_body(h_ref, w0_ref, w1_ref, w2_ref, w3_ref,
              zq_ref, sids_ref, res_ref, quant_ref, wsq_ref, wcat_ref):
    r = h_ref[...]
    bB, D = r.shape
    K = w0_ref.shape[0]
    iota_k = jax.lax.broadcasted_iota(jnp.int32, (1, K), 1)

    @pl.when(pl.program_id(0) == 0)
    def _prep():
        for lvl, w_ref in enumerate((w0_ref, w1_ref, w2_ref, w3_ref)):
            W = w_ref[...]
            wsq_ref[lvl, :] = jnp.sum(W * W, axis=1)
            w_hi = W.astype(jnp.bfloat16).astype(jnp.float32)
            wcat_ref[lvl, :, :D] = w_hi
            wcat_ref[lvl, :, D:] = W - w_hi

    zq = jnp.zeros_like(r)
    idx_cols = []
    for lvl, w_ref in enumerate((w0_ref, w1_ref, w2_ref, w3_ref)):
        W = w_ref[...]
        w_sq = wsq_ref[lvl, :][None, :]
        r_sq = jnp.sum(r * r, axis=1, keepdims=True)
        s = jax.lax.dot_general(r, W, (((1,), (1,)), ((), ())),
                                preferred_element_type=jnp.float32)
        dist = r_sq - 2.0 * s + w_sq
        m = jnp.min(dist, axis=1, keepdims=True)
        idx = jnp.min(jnp.where(dist == m, iota_k, K), axis=1)

        onehot = (iota_k == idx[:, None]).astype(jnp.float32)
        q_cat = jax.lax.dot_general(onehot, wcat_ref[lvl],
                                    (((1,), (0,)), ((), ())),
                                    preferred_element_type=jnp.float32)
        q = q_cat[:, :D] + q_cat[:, D:]

        res_ref[:, lvl, :] = r
        quant_ref[:, lvl, :] = q
        idx_cols.append(idx)
        zq = zq + q
        r = r - q

    sids_ref[...] = jnp.stack(idx_cols, axis=1)
    zq_ref[...] = zq


@jax.jit
def _rvq(h, cb0, cb1, cb2, cb3):
    B, D = h.shape
    K = cb0.shape[0]
    bB = 256
    grid = (B // bB,)
    cb_spec = pl.BlockSpec((K, D), lambda i: (0, 0))
    out_shapes = (
        jax.ShapeDtypeStruct((B, D), jnp.float32),
        jax.ShapeDtypeStruct((B, N_LEVELS), jnp.int32),
        jax.ShapeDtypeStruct((B, N_LEVELS, D), jnp.float32),
        jax.ShapeDtypeStruct((B, N_LEVELS, D), jnp.float32),
    )
    out_specs = (
        pl.BlockSpec((bB, D), lambda i: (i, 0)),
        pl.BlockSpec((bB, N_LEVELS), lambda i: (i, 0)),
        pl.BlockSpec((bB, N_LEVELS, D), lambda i: (i, 0, 0)),
        pl.BlockSpec((bB, N_LEVELS, D), lambda i: (i, 0, 0)),
    )
    in_specs = [
        pl.BlockSpec((bB, D), lambda i: (i, 0)),
        cb_spec, cb_spec, cb_spec, cb_spec,
    ]
    return pl.pallas_call(
        _rvq_body,
        grid=grid,
        in_specs=in_specs,
        out_specs=out_specs,
        out_shape=out_shapes,
        scratch_shapes=[
            pltpu.VMEM((N_LEVELS, K), jnp.float32),
            pltpu.VMEM((N_LEVELS, K, 2 * D), jnp.float32),
        ],
    )(h, cb0, cb1, cb2, cb3)


def kernel(h, cb0, cb1, cb2, cb3):
    z_q, sids, residuals, quantized = _rvq(h, cb0, cb1, cb2, cb3)
    active_mask = jnp.ones((N_LEVELS,), dtype=h.dtype)
    return (z_q, sids, residuals, quantized, active_mask)


# speedup vs baseline: 1.2703x; 1.0270x over previous
import jax
import jax.numpy as jnp
from jax.experimental import pallas as pl
from jax.experimental.pallas import tpu as pltpu


N_LEVELS = 4


def _rvq_body(h_ref, w0_ref, w1_ref, w2_ref, w3_ref,
              zq_ref, sids_ref, res_ref, quant_ref, wsq_ref):
    r = h_ref[...]
    bB, D = r.shape
    K = w0_ref.shape[0]
    iota_k = jax.lax.broadcasted_iota(jnp.int32, (1, K), 1)

    @pl.when(pl.program_id(0) == 0)
    def _prep():
        for lvl, w_ref in enumerate((w0_ref, w1_ref, w2_ref, w3_ref)):
            W = w_ref[...]
            wsq_ref[lvl, :] = jnp.sum(W * W, axis=1)

    zq = jnp.zeros_like(r)
    for lvl, w_ref in enumerate((w0_ref, w1_ref, w2_ref, w3_ref)):
        W = w_ref[...]
        w_sq = wsq_ref[lvl, :][None, :]
        r_sq = jnp.sum(r * r, axis=1, keepdims=True)
        s = jax.lax.dot_general(r, W, (((1,), (1,)), ((), ())),
                                preferred_element_type=jnp.float32)
        dist = r_sq - 2.0 * s + w_sq
        m = jnp.min(dist, axis=1, keepdims=True)
        idx = jnp.min(jnp.where(dist == m, iota_k, K), axis=1)

        onehot = (iota_k == idx[:, None]).astype(jnp.float32)
        q = jax.lax.dot_general(onehot, W, (((1,), (0,)), ((), ())),
                                preferred_element_type=jnp.float32)

        res_ref[:, lvl, :] = r
        quant_ref[:, lvl, :] = q
        sids_ref[lvl, :] = idx
        zq = zq + q
        r = r - q

    zq_ref[...] = zq


@jax.jit
def _rvq(h, cb0, cb1, cb2, cb3):
    B, D = h.shape
    K = cb0.shape[0]
    bB = 256
    grid = (B // bB,)
    cb_spec = pl.BlockSpec((K, D), lambda i: (0, 0))
    out_shapes = (
        jax.ShapeDtypeStruct((B, D), jnp.float32),
        jax.ShapeDtypeStruct((N_LEVELS, B), jnp.int32),
        jax.ShapeDtypeStruct((B, N_LEVELS, D), jnp.float32),
        jax.ShapeDtypeStruct((B, N_LEVELS, D), jnp.float32),
    )
    out_specs = (
        pl.BlockSpec((bB, D), lambda i: (i, 0)),
        pl.BlockSpec((N_LEVELS, bB), lambda i: (0, i)),
        pl.BlockSpec((bB, N_LEVELS, D), lambda i: (i, 0, 0)),
        pl.BlockSpec((bB, N_LEVELS, D), lambda i: (i, 0, 0)),
    )
    in_specs = [
        pl.BlockSpec((bB, D), lambda i: (i, 0)),
        cb_spec, cb_spec, cb_spec, cb_spec,
    ]
    return pl.pallas_call(
        _rvq_body,
        grid=grid,
        in_specs=in_specs,
        out_specs=out_specs,
        out_shape=out_shapes,
        scratch_shapes=[pltpu.VMEM((N_LEVELS, K), jnp.float32)],
    )(h, cb0, cb1, cb2, cb3)


def kernel(h, cb0, cb1, cb2, cb3):
    z_q, sids_t, residuals, quantized = _rvq(h, cb0, cb1, cb2, cb3)
    active_mask = jnp.ones((N_LEVELS,), dtype=h.dtype)
    return (z_q, sids_t.T, residuals, quantized, active_mask)
